# Initial kernel scaffold; baseline (speedup 1.0000x reference)
import jax
import jax.numpy as jnp
from jax.experimental import pallas as pl


def kernel(x, pos, sampled_idx, W1, b1, gamma1, beta1, W2, b2, gamma2, beta2):
    raise NotImplementedError("write your pallas kernel here")


# speedup vs baseline: 14.2293x; 14.2293x over previous
import functools

import jax
import jax.numpy as jnp
from jax import lax
from jax.experimental import pallas as pl
from jax.experimental.pallas import tpu as pltpu

B, N, FEAT, PDIM, M, K, CIN, COUT = 8, 2048, 253, 3, 512, 16, 256, 256


def _knn_avg_body(q_ref, post_ref, feat_ref, avg_ref, d_ref, s_ref):
    q = q_ref[0]
    pt = post_ref[0]
    dx = q[:, 0:1] - pt[0:1, :]
    dy = q[:, 1:2] - pt[1:2, :]
    dz = q[:, 2:3] - pt[2:3, :]
    d_ref[...] = (dx * dx + dy * dy) + dz * dz
    s_ref[...] = jnp.zeros((M, N), jnp.float32)
    iota = lax.broadcasted_iota(jnp.int32, (M, N), 1)
    for _ in range(K):
        d = d_ref[...]
        v = jnp.min(d, axis=1, keepdims=True)
        key = jnp.where(d == v, iota, N)
        j = jnp.min(key, axis=1, keepdims=True)
        hot = iota == j
        s_ref[...] = jnp.where(hot, 1.0, s_ref[...])
        d_ref[...] = jnp.where(hot, jnp.inf, d)
    avg = lax.dot_general(
        s_ref[...], feat_ref[0],
        (((1,), (0,)), ((), ())),
        precision=lax.Precision.HIGHEST,
        preferred_element_type=jnp.float32,
    )
    avg_ref[0] = avg * (1.0 / K)


def _mlp_body(avg_ref, w1t_ref, b1_ref, g1_ref, be1_ref, w2t_ref, b2_ref,
              g2_ref, be2_ref, out_ref):
    h = lax.dot_general(
        avg_ref[...], w1t_ref[...], (((1,), (0,)), ((), ())),
        precision=lax.Precision.HIGHEST, preferred_element_type=jnp.float32,
    ) + b1_ref[...]
    mu = jnp.mean(h, axis=0, keepdims=True)
    var = jnp.mean((h - mu) ** 2, axis=0, keepdims=True)
    h = (h - mu) / jnp.sqrt(var + 1e-5) * g1_ref[...] + be1_ref[...]
    h = jnp.maximum(h, 0.0)
    h = lax.dot_general(
        h, w2t_ref[...], (((1,), (0,)), ((), ())),
        precision=lax.Precision.HIGHEST, preferred_element_type=jnp.float32,
    ) + b2_ref[...]
    mu = jnp.mean(h, axis=0, keepdims=True)
    var = jnp.mean((h - mu) ** 2, axis=0, keepdims=True)
    out_ref[...] = (h - mu) / jnp.sqrt(var + 1e-5) * g2_ref[...] + be2_ref[...]


@functools.partial(jax.jit, static_argnames=("interpret",))
def kernel(x, pos, sampled_idx, W1, b1, gamma1, beta1, W2, b2, gamma2, beta2,
           interpret=False):
    pos_flat = pos.reshape(B * N, PDIM)
    q = pos_flat[sampled_idx].reshape(B, M, PDIM)
    pos_t = jnp.transpose(pos, (0, 2, 1))
    feat = jnp.concatenate([x, pos], axis=-1)

    avg = pl.pallas_call(
        _knn_avg_body,
        grid=(B,),
        in_specs=[
            pl.BlockSpec((1, M, PDIM), lambda b: (b, 0, 0)),
            pl.BlockSpec((1, PDIM, N), lambda b: (b, 0, 0)),
            pl.BlockSpec((1, N, CIN), lambda b: (b, 0, 0)),
        ],
        out_specs=pl.BlockSpec((1, M, CIN), lambda b: (b, 0, 0)),
        out_shape=jax.ShapeDtypeStruct((B, M, CIN), jnp.float32),
        scratch_shapes=[
            pltpu.VMEM((M, N), jnp.float32),
            pltpu.VMEM((M, N), jnp.float32),
        ],
        interpret=interpret,
    )(q, pos_t, feat)

    out = pl.pallas_call(
        _mlp_body,
        out_shape=jax.ShapeDtypeStruct((B * M, COUT), jnp.float32),
        interpret=interpret,
    )(avg.reshape(B * M, CIN), W1.T, b1.reshape(1, COUT),
      gamma1.reshape(1, COUT), beta1.reshape(1, COUT), W2.T,
      b2.reshape(1, COUT), gamma2.reshape(1, COUT), beta2.reshape(1, COUT))

    return out.reshape(B, M, COUT)
